# initial kernel scaffold (unmeasured)
import jax
import jax.numpy as jnp
from jax import lax
from jax.experimental import pallas as pl
from jax.experimental.pallas import tpu as pltpu

B, SQ, SKV, H, D = 8, 8, 1024, 16, 128
SCALE = D ** -0.5


def kernel(Q, K, V):
    def body(q_ref, k_ref, v_ref, out_ref, ml_ref, o_recv, ml_recv,
             send_sems, recv_sems):
        b = pl.program_id(0)
        my_x = lax.axis_index("x")
        my_y = lax.axis_index("y")
        my_z = lax.axis_index("z")
        partner = (my_x, my_y, 1 - my_z)

        @pl.when(b == 0)
        def _():
            barrier = pltpu.get_barrier_semaphore()
            pl.semaphore_signal(
                barrier, inc=1, device_id=partner,
                device_id_type=pl.DeviceIdType.MESH,
            )
            pl.semaphore_wait(barrier, 1)

        for h in range(H):
            q = q_ref[0, :, h, :].astype(jnp.bfloat16)
            k = k_ref[0, :, h, :].astype(jnp.bfloat16)
            v = v_ref[0, :, h, :].astype(jnp.bfloat16)
            s = lax.dot_general(
                q, k, (((1,), (1,)), ((), ())),
                preferred_element_type=jnp.float32,
            ) * SCALE
            m = jnp.max(s, axis=1, keepdims=True)
            p = jnp.exp(s - m)
            l = jnp.sum(p, axis=1, keepdims=True)
            o = lax.dot_general(
                p.astype(jnp.bfloat16), v, (((1,), (0,)), ((), ())),
                preferred_element_type=jnp.float32,
            )
            out_ref[pl.ds(b, 1), :, h, :] = o[None]
            ml_ref[0, pl.ds(b, 1), :, h] = m[:, 0][None]
            ml_ref[1, pl.ds(b, 1), :, h] = l[:, 0][None]

        @pl.when(b == B - 1)
        def _():
            rdma_o = pltpu.make_async_remote_copy(
                src_ref=out_ref, dst_ref=o_recv,
                send_sem=send_sems.at[0], recv_sem=recv_sems.at[0],
                device_id=partner, device_id_type=pl.DeviceIdType.MESH,
            )
            rdma_ml = pltpu.make_async_remote_copy(
                src_ref=ml_ref, dst_ref=ml_recv,
                send_sem=send_sems.at[1], recv_sem=recv_sems.at[1],
                device_id=partner, device_id_type=pl.DeviceIdType.MESH,
            )
            rdma_o.start()
            rdma_ml.start()
            rdma_o.wait()
            rdma_ml.wait()

            m1 = ml_ref[0]
            l1 = ml_ref[1]
            m2 = ml_recv[0]
            l2 = ml_recv[1]
            mx = jnp.maximum(m1, m2)
            a1 = jnp.exp(m1 - mx)
            a2 = jnp.exp(m2 - mx)
            lsum = a1 * l1 + a2 * l2
            o1 = out_ref[...]
            o2 = o_recv[...]
            out_ref[...] = (
                a1[..., None] * o1 + a2[..., None] * o2
            ) / lsum[..., None]

    return pl.pallas_call(
        body,
        grid=(B,),
        out_shape=jax.ShapeDtypeStruct((B, SQ, H, D), jnp.float32),
        in_specs=[
            pl.BlockSpec((1, SQ, H, D), lambda b: (b, 0, 0, 0)),
            pl.BlockSpec((1, SKV, H, D), lambda b: (b, 0, 0, 0)),
            pl.BlockSpec((1, SKV, H, D), lambda b: (b, 0, 0, 0)),
        ],
        out_specs=pl.BlockSpec((B, SQ, H, D), lambda b: (0, 0, 0, 0)),
        scratch_shapes=[
            pltpu.VMEM((2, B, SQ, H), jnp.float32),
            pltpu.VMEM((B, SQ, H, D), jnp.float32),
            pltpu.VMEM((2, B, SQ, H), jnp.float32),
            pltpu.SemaphoreType.DMA((2,)),
            pltpu.SemaphoreType.DMA((2,)),
        ],
        compiler_params=pltpu.CompilerParams(collective_id=0),
    )(Q, K, V)


# baseline (device time: 160570 ns/iter reference)
import jax
import jax.numpy as jnp
from jax import lax
from jax.experimental import pallas as pl
from jax.experimental.pallas import tpu as pltpu

B, SQ, SKV, H, D = 8, 8, 1024, 16, 128
SCALE = D ** -0.5


def kernel(Q, K, V):
    def body(q_ref, k_ref, v_ref, out_ref, ml_ref, o_recv, ml_recv,
             send_sems, recv_sems):
        b = pl.program_id(0)
        my_x = lax.axis_index("x")
        my_y = lax.axis_index("y")
        my_z = lax.axis_index("z")
        partner = (my_x, my_y, 1 - my_z)

        @pl.when(b == 0)
        def _():
            barrier = pltpu.get_barrier_semaphore()
            pl.semaphore_signal(
                barrier, inc=1, device_id=partner,
                device_id_type=pl.DeviceIdType.MESH,
            )
            pl.semaphore_wait(barrier, 1)

        for h in range(H):
            q = q_ref[0, :, h, :].astype(jnp.bfloat16)
            k = k_ref[0, :, h, :].astype(jnp.bfloat16)
            v = v_ref[0, :, h, :].astype(jnp.bfloat16)
            s = lax.dot_general(
                q, k, (((1,), (1,)), ((), ())),
                preferred_element_type=jnp.float32,
            ) * SCALE
            m = jnp.max(s, axis=1, keepdims=True)
            p = jnp.exp(s - m)
            l = jnp.sum(p, axis=1, keepdims=True)
            o = lax.dot_general(
                p.astype(jnp.bfloat16), v, (((1,), (0,)), ((), ())),
                preferred_element_type=jnp.float32,
            )
            out_ref[pl.ds(b, 1), :, h, :] = o[None]
            ml_ref[0, pl.ds(b, 1), :, h] = m[:, 0][None]
            ml_ref[1, pl.ds(b, 1), :, h] = l[:, 0][None]

        @pl.when(b == B - 1)
        def _():
            rdma_o = pltpu.make_async_remote_copy(
                src_ref=out_ref, dst_ref=o_recv,
                send_sem=send_sems.at[0], recv_sem=recv_sems.at[0],
                device_id=partner, device_id_type=pl.DeviceIdType.MESH,
            )
            rdma_ml = pltpu.make_async_remote_copy(
                src_ref=ml_ref, dst_ref=ml_recv,
                send_sem=send_sems.at[1], recv_sem=recv_sems.at[1],
                device_id=partner, device_id_type=pl.DeviceIdType.MESH,
            )
            rdma_o.start()
            rdma_ml.start()
            rdma_o.wait()
            rdma_ml.wait()

            m1 = ml_ref[0]
            l1 = ml_ref[1]
            m2 = ml_recv[0]
            l2 = ml_recv[1]
            mx = jnp.maximum(m1, m2)
            a1 = jnp.exp(m1 - mx)
            a2 = jnp.exp(m2 - mx)
            lsum = a1 * l1 + a2 * l2
            o1 = out_ref[...]
            o2 = o_recv[...]
            out_ref[...] = (
                a1[..., None] * o1 + a2[..., None] * o2
            ) / lsum[..., None]

    return pl.pallas_call(
        body,
        grid=(B,),
        out_shape=jax.ShapeDtypeStruct((B, SQ, H, D), jnp.float32),
        in_specs=[
            pl.BlockSpec((1, SQ, H, D), lambda b: (b, 0, 0, 0)),
            pl.BlockSpec((1, SKV, H, D), lambda b: (b, 0, 0, 0)),
            pl.BlockSpec((1, SKV, H, D), lambda b: (b, 0, 0, 0)),
        ],
        out_specs=pl.BlockSpec((B, SQ, H, D), lambda b: (0, 0, 0, 0)),
        scratch_shapes=[
            pltpu.VMEM((2, B, SQ, H), jnp.float32),
            pltpu.VMEM((B, SQ, H, D), jnp.float32),
            pltpu.VMEM((2, B, SQ, H), jnp.float32),
            pltpu.SemaphoreType.DMA((2,)),
            pltpu.SemaphoreType.DMA((2,)),
        ],
        compiler_params=pltpu.CompilerParams(
            collective_id=0, vmem_limit_bytes=64 * 1024 * 1024,
        ),
    )(Q, K, V)
